# Initial kernel scaffold; baseline (speedup 1.0000x reference)
import jax
import jax.numpy as jnp
from jax.experimental import pallas as pl


def kernel(x, tables):
    raise NotImplementedError("write your pallas kernel here")


# speedup vs baseline: 7.4481x; 7.4481x over previous
import functools

import jax
import jax.numpy as jnp
from jax import lax
from jax.experimental import pallas as pl
from jax.experimental.pallas import tpu as pltpu
from jax.experimental.pallas import tpu_sc as plsc

B, L, NF = 16384, 50, 5
VOCAB, DIM = 100000, 32
N = B * L

NC, NS, LANES = 2, 16, 16
NW = NC * NS
N_PER_W = N // NW

C = 512
G_IDX = 80
N_GROUPS = (C * NF) // G_IDX
N_CHUNKS = N_PER_W // C


def _body(x_hbm, off_hbm, tab_hbm, out_hbm, xv, rows, outv, offv, sem):
    wid = lax.axis_index("s") * NC + lax.axis_index("c")
    base = wid * N_PER_W

    pltpu.sync_copy(off_hbm, offv)

    def chunk_body(k, _):
        n0 = base + k * C
        pltpu.sync_copy(x_hbm.at[pl.ds(n0 * NF, NF * C)], xv)

        def off_body(j, _):
            for t in range(NF):
                s = j * (NF * LANES) + t * LANES
                xv[pl.ds(s, LANES)] = xv[pl.ds(s, LANES)] + offv[pl.ds(t * LANES, LANES)]
            return _

        lax.fori_loop(0, (NF * C) // (NF * LANES), off_body, None)

        def gather_body(g, _):
            pltpu.async_copy(
                tab_hbm.at[xv.at[pl.ds(g * G_IDX, G_IDX)]],
                rows.at[pl.ds(g * G_IDX, G_IDX), :],
                sem,
            )
            return _

        lax.fori_loop(0, N_GROUPS, gather_body, None)
        pltpu.make_async_copy(tab_hbm.at[pl.ds(0, NF * C)], rows, sem).wait()

        def red_body(c, _):
            r0 = c * NF
            lo = rows[r0, pl.ds(0, LANES)]
            hi = rows[r0, pl.ds(LANES, LANES)]
            for t in range(1, NF):
                lo = lo + rows[r0 + t, pl.ds(0, LANES)]
                hi = hi + rows[r0 + t, pl.ds(LANES, LANES)]
            outv[c, pl.ds(0, LANES)] = lo
            outv[c, pl.ds(LANES, LANES)] = hi
            return _

        lax.fori_loop(0, C, red_body, None)

        pltpu.sync_copy(outv, out_hbm.at[pl.ds(n0, C), :])
        return _

    lax.fori_loop(0, N_CHUNKS, chunk_body, None)


@jax.jit
def _run(x_flat, offpat, tab_flat):
    mesh = plsc.VectorSubcoreMesh(core_axis_name="c", subcore_axis_name="s")
    return pl.kernel(
        _body,
        mesh=mesh,
        compiler_params=pltpu.CompilerParams(use_tc_tiling_on_sc=False),
        out_type=jax.ShapeDtypeStruct((N, DIM), jnp.float32),
        scratch_types=[
            pltpu.VMEM((NF * C,), jnp.int32),
            pltpu.VMEM((NF * C, DIM), jnp.float32),
            pltpu.VMEM((C, DIM), jnp.float32),
            pltpu.VMEM((NF * LANES,), jnp.int32),
            pltpu.SemaphoreType.DMA,
        ],
    )(x_flat, offpat, tab_flat)


def kernel(x, tables):
    x_flat = x.reshape(-1)
    tab_flat = tables.reshape(NF * VOCAB, DIM)
    offpat = jnp.tile(jnp.arange(NF, dtype=jnp.int32) * VOCAB, LANES)
    out = _run(x_flat, offpat, tab_flat)
    return out.reshape(B, L, DIM)


# speedup vs baseline: 8.7678x; 1.1772x over previous
import functools

import jax
import jax.numpy as jnp
from jax import lax
from jax.experimental import pallas as pl
from jax.experimental.pallas import tpu as pltpu
from jax.experimental.pallas import tpu_sc as plsc

B, L, NF = 16384, 50, 5
VOCAB, DIM = 100000, 32
N = B * L

NC, NS, LANES = 2, 16, 16
NW = NC * NS
N_PER_W = N // NW

C = 256
NI = NF * C
G_IDX = 128
N_GROUPS = NI // G_IDX
N_CHUNKS = N_PER_W // C


def _body(x_hbm, off_hbm, tab_hbm, out_hbm,
          xs_a, xs_b, xv_a, xv_b, rows_a, rows_b, outv_a, outv_b, offv,
          sem_xa, sem_xb, sem_ga, sem_gb, sem_oa, sem_ob):
    wid = lax.axis_index("s") * NC + lax.axis_index("c")
    base = wid * N_PER_W

    pltpu.sync_copy(off_hbm, offv)

    def xload(chunk, xs, sem):
        pltpu.async_copy(x_hbm.at[pl.ds((base + chunk * C) * NF, NI)], xs, sem)

    def xwait(xs, sem):
        pltpu.make_async_copy(x_hbm.at[pl.ds(0, NI)], xs, sem).wait()

    def offadd_and_fire(xs, xv, rows, sem):
        for j in range(NI // (NF * LANES)):
            for t in range(NF):
                s = j * (NF * LANES) + t * LANES
                xv[pl.ds(s, LANES)] = xs[pl.ds(s, LANES)] + offv[pl.ds(t * LANES, LANES)]
        for g in range(N_GROUPS):
            pltpu.async_copy(
                tab_hbm.at[xv.at[pl.ds(g * G_IDX, G_IDX)]],
                rows.at[pl.ds(g * G_IDX, G_IDX), :],
                sem,
            )

    def gwait(rows, sem):
        pltpu.make_async_copy(tab_hbm.at[pl.ds(0, NI)], rows, sem).wait()

    def reduce(rows, outv):
        @plsc.parallel_loop(0, C, unroll=4)
        def red_body(c):
            r0 = c * NF
            lo = rows[r0, pl.ds(0, LANES)]
            hi = rows[r0, pl.ds(LANES, LANES)]
            for t in range(1, NF):
                lo = lo + rows[r0 + t, pl.ds(0, LANES)]
                hi = hi + rows[r0 + t, pl.ds(LANES, LANES)]
            outv[c, pl.ds(0, LANES)] = lo
            outv[c, pl.ds(LANES, LANES)] = hi

    def owrite(chunk, outv, sem):
        pltpu.async_copy(outv, out_hbm.at[pl.ds(base + chunk * C, C), :], sem)

    def owait(outv, sem):
        pltpu.make_async_copy(outv, out_hbm.at[pl.ds(base, C), :], sem).wait()

    xload(0, xs_a, sem_xa)
    xwait(xs_a, sem_xa)
    offadd_and_fire(xs_a, xv_a, rows_a, sem_ga)
    xload(1, xs_b, sem_xb)

    def loop(kk, _):
        c0 = 2 * kk
        xwait(xs_b, sem_xb)
        offadd_and_fire(xs_b, xv_b, rows_b, sem_gb)

        @pl.when(c0 + 2 < N_CHUNKS)
        def _():
            xload(c0 + 2, xs_a, sem_xa)

        gwait(rows_a, sem_ga)

        @pl.when(kk > 0)
        def _():
            owait(outv_a, sem_oa)

        reduce(rows_a, outv_a)
        owrite(c0, outv_a, sem_oa)

        @pl.when(c0 + 2 < N_CHUNKS)
        def _():
            xwait(xs_a, sem_xa)
            offadd_and_fire(xs_a, xv_a, rows_a, sem_ga)
            xload(c0 + 3, xs_b, sem_xb)

        gwait(rows_b, sem_gb)

        @pl.when(kk > 0)
        def _():
            owait(outv_b, sem_ob)

        reduce(rows_b, outv_b)
        owrite(c0 + 1, outv_b, sem_ob)
        return _

    lax.fori_loop(0, N_CHUNKS // 2, loop, None)
    owait(outv_a, sem_oa)
    owait(outv_b, sem_ob)


@jax.jit
def _run(x_flat, offpat, tab_flat):
    mesh = plsc.VectorSubcoreMesh(core_axis_name="c", subcore_axis_name="s")
    return pl.kernel(
        _body,
        mesh=mesh,
        compiler_params=pltpu.CompilerParams(use_tc_tiling_on_sc=False),
        out_type=jax.ShapeDtypeStruct((N, DIM), jnp.float32),
        scratch_types=[
            pltpu.VMEM((NI,), jnp.int32),
            pltpu.VMEM((NI,), jnp.int32),
            pltpu.VMEM((NI,), jnp.int32),
            pltpu.VMEM((NI,), jnp.int32),
            pltpu.VMEM((NI, DIM), jnp.float32),
            pltpu.VMEM((NI, DIM), jnp.float32),
            pltpu.VMEM((C, DIM), jnp.float32),
            pltpu.VMEM((C, DIM), jnp.float32),
            pltpu.VMEM((NF * LANES,), jnp.int32),
            pltpu.SemaphoreType.DMA,
            pltpu.SemaphoreType.DMA,
            pltpu.SemaphoreType.DMA,
            pltpu.SemaphoreType.DMA,
            pltpu.SemaphoreType.DMA,
            pltpu.SemaphoreType.DMA,
        ],
    )(x_flat, offpat, tab_flat)


def kernel(x, tables):
    x_flat = x.reshape(-1)
    tab_flat = tables.reshape(NF * VOCAB, DIM)
    offpat = jnp.tile(jnp.arange(NF, dtype=jnp.int32) * VOCAB, LANES)
    out = _run(x_flat, offpat, tab_flat)
    return out.reshape(B, L, DIM)
